# Initial kernel scaffold; baseline (speedup 1.0000x reference)
import jax
import jax.numpy as jnp
from jax.experimental import pallas as pl


def kernel(user_embed, item_embed, edge_vals, edge_index, users, pos_items, neg_items):
    raise NotImplementedError("write your pallas kernel here")


# speedup vs baseline: 1.0213x; 1.0213x over previous
import functools

import jax
import jax.numpy as jnp
import numpy as np
from jax.experimental import pallas as pl
from jax.experimental.pallas import tpu as pltpu

N_USERS = 25000
N_ITEMS = 25000
N_NODES = N_USERS + N_ITEMS
EMB = 64
N_EDGES = 800000
N_HOPS = 3
LAYER_CL = 1
EPS = 0.2
TEMP = 0.2
CL_RATE = 0.005
DECAY = 0.0001
BATCH = 4096


def _precompute_noise() -> np.ndarray:
    key = jax.random.key(42)
    out = []
    for _ in range(N_HOPS):
        key, sub = jax.random.split(key)
        r = np.asarray(jax.random.uniform(sub, (N_NODES, EMB), dtype=jnp.float32))
        n = np.sqrt((r * r).sum(axis=-1, keepdims=True))
        out.append(r / np.maximum(n, 1e-12) * EPS)
    return np.stack(out)


_NOISE = _precompute_noise()


def _normalize(x, axis=-1):
    n = jnp.linalg.norm(x, axis=axis, keepdims=True)
    return x / jnp.maximum(n, 1e-12)


_ROWS_BLK = 512


def _nce_body(v1_blk, v2_full, v3_full, v2_blk, v3_blk, maskf_full, out_ref):
    i = pl.program_id(0)
    a = v1_blk[...]
    v2 = v2_full[...]
    v3 = v3_full[...]
    mrow = maskf_full[0, pl.ds(i * _ROWS_BLK, _ROWS_BLK)]
    mcol = maskf_full[0, :]

    dim = (((1,), (1,)), ((), ()))
    s12 = jax.lax.dot_general(a, v2, dim, preferred_element_type=jnp.float32)
    ttl12 = jnp.sum(jnp.exp(s12 / TEMP) * mcol[None, :], axis=1)
    s13 = jax.lax.dot_general(a, v3, dim, preferred_element_type=jnp.float32)
    ttl13 = jnp.sum(jnp.exp(s13 / TEMP) * mcol[None, :], axis=1)

    b2 = v2_blk[...]
    b3 = v3_blk[...]
    pos12 = jnp.exp(jnp.sum(a * b2, axis=1) / TEMP)
    pos13 = jnp.exp(jnp.sum(a * b3, axis=1) / TEMP)

    term = -jnp.log(pos12 / ttl12 + 1e-10) - jnp.log(pos13 / ttl13 + 1e-10)
    part = jnp.sum(jnp.where(mrow > 0.5, term, 0.0)).reshape(1, 1)

    @pl.when(i == 0)
    def _init():
        out_ref[...] = jnp.zeros((1, 1), jnp.float32)

    out_ref[...] += part


def _info_nce_pallas(view1, view2, view3, maskf):
    m = maskf[:, None]
    v1 = _normalize(view1, axis=1) * m
    v2 = _normalize(view2, axis=1) * m
    v3 = _normalize(view3, axis=1) * m
    grid = BATCH // _ROWS_BLK
    out = pl.pallas_call(
        _nce_body,
        grid=(grid,),
        in_specs=[
            pl.BlockSpec((_ROWS_BLK, EMB), lambda i: (i, 0)),
            pl.BlockSpec((BATCH, EMB), lambda i: (0, 0)),
            pl.BlockSpec((BATCH, EMB), lambda i: (0, 0)),
            pl.BlockSpec((_ROWS_BLK, EMB), lambda i: (i, 0)),
            pl.BlockSpec((_ROWS_BLK, EMB), lambda i: (i, 0)),
            pl.BlockSpec((1, BATCH), lambda i: (0, 0)),
        ],
        out_specs=pl.BlockSpec((1, 1), lambda i: (0, 0)),
        out_shape=jax.ShapeDtypeStruct((1, 1), jnp.float32),
    )(v1, v2, v3, v2, v3, maskf[None, :])
    return out[0, 0]


def kernel(user_embed, item_embed, edge_vals, edge_index, users, pos_items, neg_items):
    ego = jnp.concatenate([user_embed, item_embed], axis=0)
    noise = jnp.asarray(_NOISE)
    src = edge_index[1]
    dst = edge_index[0]
    hops = []
    for k in range(N_HOPS):
        msg = edge_vals[:, None] * jnp.take(ego, src, axis=0)
        ego = jnp.zeros((N_NODES, EMB), dtype=ego.dtype).at[dst].add(msg)
        ego = ego + jnp.sign(ego) * noise[k]
        hops.append(ego)
    final = (hops[0] + hops[1] + hops[2]) * (1.0 / 3.0)
    cl, cl1 = hops[0], hops[1]

    rec_user_emb, rec_item_emb = final[:N_USERS], final[N_USERS:]
    user_emb = jnp.take(rec_user_emb, users, axis=0)
    pos_item_emb = jnp.take(rec_item_emb, pos_items, axis=0)
    neg_item_emb = jnp.take(rec_item_emb, neg_items, axis=0)
    pos_score = jnp.sum(user_emb * pos_item_emb, axis=1)
    neg_score = jnp.sum(user_emb * neg_item_emb, axis=1)
    rec_loss = jnp.mean(-jnp.log(1e-05 + jax.nn.sigmoid(pos_score - neg_score)))
    emb_loss = (jnp.linalg.norm(user_emb) + jnp.linalg.norm(pos_item_emb)) * DECAY

    u_idx = jnp.unique(users, size=BATCH, fill_value=0)
    i_idx = jnp.unique(pos_items, size=BATCH, fill_value=0)
    su = jnp.sort(users)
    n_u = 1 + jnp.sum((su[1:] != su[:-1]).astype(jnp.int32))
    si = jnp.sort(pos_items)
    n_i = 1 + jnp.sum((si[1:] != si[:-1]).astype(jnp.int32))
    u_maskf = (jnp.arange(BATCH) < n_u).astype(jnp.float32)
    i_maskf = (jnp.arange(BATCH) < n_i).astype(jnp.float32)

    cl_user_emb, cl_item_emb = cl[:N_USERS], cl[N_USERS:]
    cl1_user_emb, cl1_item_emb = cl1[:N_USERS], cl1[N_USERS:]
    user_cl = _info_nce_pallas(rec_user_emb[u_idx], cl_user_emb[u_idx],
                               cl1_user_emb[u_idx], u_maskf) / n_u.astype(jnp.float32)
    item_cl = _info_nce_pallas(rec_item_emb[i_idx], cl_item_emb[i_idx],
                               cl1_item_emb[i_idx], i_maskf) / n_i.astype(jnp.float32)
    cl_loss = CL_RATE * (user_cl + item_cl)
    return rec_loss + emb_loss + cl_loss
